# Initial kernel scaffold; baseline (speedup 1.0000x reference)
import functools

import jax
import jax.numpy as jnp
from jax import lax
from jax.experimental import pallas as pl
from jax.experimental.pallas import tpu as pltpu
from jax.experimental.pallas import tpu_sc as plsc

N_NODES = 10000
N_EDGES = 320000
D_FEAT = 128
HIDDEN = 256
D_OUT = 128

NT = 16
CH = 128
CHUNKS = 158
EDGES_PAD = NT * CHUNKS * CH
NPAD = 10240
RPT = NPAD // NT
DUMMY = N_NODES
DEGW = 16

_f32 = jnp.float32
_mesh = plsc.VectorSubcoreMesh(core_axis_name="c", subcore_axis_name="s")


def _sds(shape):
    return jax.ShapeDtypeStruct(shape, _f32)


@functools.partial(
    pl.kernel, mesh=_mesh,
    out_type=(_sds((NPAD, DEGW)), _sds((NPAD, DEGW))),
    scratch_types=[
        pltpu.VMEM((CHUNKS, CH), jnp.int32),
        pltpu.VMEM((CH, DEGW), _f32),
        pltpu.VMEM_SHARED((NPAD, DEGW), _f32),
    ])
def _sc_degree(dsti_h, ones_h, z_h, out0, out1, dstv, onesv, table):
    c = lax.axis_index("c")
    s = lax.axis_index("s")
    pltpu.sync_copy(dsti_h.at[s], dstv)
    pltpu.sync_copy(ones_h, onesv)
    pltpu.sync_copy(z_h, table.at[pl.ds(s * RPT, RPT)])
    plsc.subcore_barrier()
    half = CHUNKS // 2
    base = c * half

    @pl.loop(0, half)
    def _(j):
        pltpu.sync_copy(onesv, table.at[dstv.at[base + j]], add=True)

    plsc.subcore_barrier()

    @pl.when(c == 0)
    def _():
        pltpu.sync_copy(table.at[pl.ds(s * RPT, RPT)], out0.at[pl.ds(s * RPT, RPT)])

    @pl.when(c == 1)
    def _():
        pltpu.sync_copy(table.at[pl.ds(s * RPT, RPT)], out1.at[pl.ds(s * RPT, RPT)])


def _make_sc_agg(D):
    @functools.partial(
        pl.kernel, mesh=_mesh,
        out_type=(_sds((NPAD, D)), _sds((NPAD, D))),
        scratch_types=[
            pltpu.VMEM((CHUNKS, CH), jnp.int32),
            pltpu.VMEM((CHUNKS, CH), jnp.int32),
            pltpu.VMEM((CH, D), _f32),
            pltpu.VMEM_SHARED((NPAD, D), _f32),
        ])
    def kagg(h0, h1, srci, dsti, z_h, out0, out1, srcv, dstv, gbuf, table):
        c = lax.axis_index("c")
        s = lax.axis_index("s")
        pltpu.sync_copy(srci.at[s], srcv)
        pltpu.sync_copy(dsti.at[s], dstv)
        pltpu.sync_copy(z_h, table.at[pl.ds(s * RPT, RPT)])
        plsc.subcore_barrier()

        def run(h):
            @pl.loop(0, CHUNKS)
            def _(j):
                pltpu.sync_copy(h.at[srcv.at[j]], gbuf)
                pltpu.sync_copy(gbuf, table.at[dstv.at[j]], add=True)

        @pl.when(c == 0)
        def _():
            run(h0)

        @pl.when(c == 1)
        def _():
            run(h1)

        plsc.subcore_barrier()

        @pl.when(c == 0)
        def _():
            pltpu.sync_copy(table.at[pl.ds(s * RPT, RPT)], out0.at[pl.ds(s * RPT, RPT)])

        @pl.when(c == 1)
        def _():
            pltpu.sync_copy(table.at[pl.ds(s * RPT, RPT)], out1.at[pl.ds(s * RPT, RPT)])

    return kagg


_sc_agg128 = _make_sc_agg(HIDDEN // 2)
_sc_agg64 = _make_sc_agg(D_OUT // 2)


_BLK = 1000
_GRID = N_NODES // _BLK


def _tc_mm1(x, W1):
    H2 = HIDDEN // 2

    def body(x_ref, w_ref, oa, ob):
        h = jnp.dot(x_ref[...], w_ref[...], preferred_element_type=_f32)
        oa[...] = h[:, :H2]
        ob[...] = h[:, H2:]

    return pl.pallas_call(
        body, grid=(_GRID,),
        in_specs=[pl.BlockSpec((_BLK, D_FEAT), lambda i: (i, 0)),
                  pl.BlockSpec((D_FEAT, HIDDEN), lambda i: (0, 0))],
        out_specs=[pl.BlockSpec((_BLK, H2), lambda i: (i, 0))] * 2,
        out_shape=(_sds((N_NODES, H2)),) * 2,
    )(x, W1)


def _tc_scale(h1a, h1b, dega, degb):
    H2 = HIDDEN // 2

    def body(ha, hb, da, db, dinv_ref, oa, ob):
        d = da[...][:, 0:1] + db[...][:, 0:1] + 1.0
        dv = lax.rsqrt(d)
        dinv_ref[...] = dv
        oa[...] = ha[...] * dv
        ob[...] = hb[...] * dv

    return pl.pallas_call(
        body, grid=(_GRID,),
        in_specs=[pl.BlockSpec((_BLK, H2), lambda i: (i, 0)),
                  pl.BlockSpec((_BLK, H2), lambda i: (i, 0)),
                  pl.BlockSpec((_BLK, DEGW), lambda i: (i, 0)),
                  pl.BlockSpec((_BLK, DEGW), lambda i: (i, 0))],
        out_specs=[pl.BlockSpec((_BLK, 1), lambda i: (i, 0)),
                   pl.BlockSpec((_BLK, H2), lambda i: (i, 0)),
                   pl.BlockSpec((_BLK, H2), lambda i: (i, 0))],
        out_shape=(_sds((N_NODES, 1)), _sds((N_NODES, H2)), _sds((N_NODES, H2))),
    )(h1a, h1b, dega, degb)


def _tc_fuse(aa, ab, hsa, hsb, dinv, b, W, out_half):
    H2 = HIDDEN // 2
    wN = W.shape[1]

    def body(aa_r, ab_r, hsa_r, hsb_r, dv_r, b_r, w_r, oa, ob):
        dv = dv_r[...]
        z = jnp.concatenate([aa_r[...] + hsa_r[...], ab_r[...] + hsb_r[...]], axis=1)
        z = z * dv + b_r[...]
        z = jnp.maximum(z, 0.0)
        h = jnp.dot(z, w_r[...], preferred_element_type=_f32) * dv
        oa[...] = h[:, :out_half]
        ob[...] = h[:, out_half:]

    return pl.pallas_call(
        body, grid=(_GRID,),
        in_specs=[pl.BlockSpec((_BLK, H2), lambda i: (i, 0)),
                  pl.BlockSpec((_BLK, H2), lambda i: (i, 0)),
                  pl.BlockSpec((_BLK, H2), lambda i: (i, 0)),
                  pl.BlockSpec((_BLK, H2), lambda i: (i, 0)),
                  pl.BlockSpec((_BLK, 1), lambda i: (i, 0)),
                  pl.BlockSpec((1, wN), lambda i: (0, 0)),
                  pl.BlockSpec((HIDDEN, wN), lambda i: (0, 0))],
        out_specs=[pl.BlockSpec((_BLK, out_half), lambda i: (i, 0))] * 2,
        out_shape=(_sds((N_NODES, out_half)),) * 2,
    )(aa, ab, hsa, hsb, dinv, b.reshape(1, wN), W)


def _tc_final(aa, ab, hsa, hsb, dinv, b_out):
    O2 = D_OUT // 2

    def body(aa_r, ab_r, hsa_r, hsb_r, dv_r, b_r, o):
        z = jnp.concatenate([aa_r[...] + hsa_r[...], ab_r[...] + hsb_r[...]], axis=1)
        o[...] = z * dv_r[...] + b_r[...]

    return pl.pallas_call(
        body, grid=(_GRID,),
        in_specs=[pl.BlockSpec((_BLK, O2), lambda i: (i, 0)),
                  pl.BlockSpec((_BLK, O2), lambda i: (i, 0)),
                  pl.BlockSpec((_BLK, O2), lambda i: (i, 0)),
                  pl.BlockSpec((_BLK, O2), lambda i: (i, 0)),
                  pl.BlockSpec((_BLK, 1), lambda i: (i, 0)),
                  pl.BlockSpec((1, D_OUT), lambda i: (0, 0))],
        out_specs=pl.BlockSpec((_BLK, D_OUT), lambda i: (i, 0)),
        out_shape=_sds((N_NODES, D_OUT)),
    )(aa, ab, hsa, hsb, dinv, b_out.reshape(1, D_OUT))


def kernel(x, edge_index, batch, W1, b1, W2, b2, W_out, b_out):
    src = edge_index[0]
    dst = edge_index[1]
    pad = EDGES_PAD - N_EDGES
    srcp = jnp.concatenate([src, jnp.zeros((pad,), jnp.int32)]).reshape(NT, CHUNKS, CH)
    dstp = jnp.concatenate([dst, jnp.full((pad,), DUMMY, jnp.int32)]).reshape(NT, CHUNKS, CH)
    ones16 = jnp.ones((CH, DEGW), _f32)
    z16 = jnp.zeros((RPT, DEGW), _f32)
    z128 = jnp.zeros((RPT, HIDDEN // 2), _f32)
    z64 = jnp.zeros((RPT, D_OUT // 2), _f32)

    dega, degb = _sc_degree(dstp, ones16, z16)
    h1a, h1b = _tc_mm1(x, W1)
    dinv, h1sa, h1sb = _tc_scale(h1a, h1b, dega, degb)
    a1a, a1b = _sc_agg128(h1sa, h1sb, srcp, dstp, z128)
    h2sa, h2sb = _tc_fuse(a1a, a1b, h1sa, h1sb, dinv, b1, W2, HIDDEN // 2)
    a2a, a2b = _sc_agg128(h2sa, h2sb, srcp, dstp, z128)
    h3sa, h3sb = _tc_fuse(a2a, a2b, h2sa, h2sb, dinv, b2, W_out, D_OUT // 2)
    a3a, a3b = _sc_agg64(h3sa, h3sb, srcp, dstp, z64)
    return _tc_final(a3a, a3b, h3sa, h3sb, dinv, b_out)


# speedup vs baseline: 4.3626x; 4.3626x over previous
import functools

import jax
import jax.numpy as jnp
from jax import lax
from jax.experimental import pallas as pl
from jax.experimental.pallas import tpu as pltpu
from jax.experimental.pallas import tpu_sc as plsc

N_NODES = 10000
N_EDGES = 320000
D_FEAT = 128
HIDDEN = 256
D_OUT = 128

NT = 16
CH = 128
CHUNKS = 160
EDGES_PAD = NT * CHUNKS * CH
NPAD = 10240
RPT = NPAD // NT
DUMMY = N_NODES
DEGW = 128

_f32 = jnp.float32
_mesh = plsc.VectorSubcoreMesh(core_axis_name="c", subcore_axis_name="s")


def _sds(shape):
    return jax.ShapeDtypeStruct(shape, _f32)


@functools.partial(
    pl.kernel, mesh=_mesh,
    out_type=(_sds((NPAD, DEGW)), _sds((NPAD, DEGW))),
    scratch_types=[
        pltpu.VMEM((CHUNKS // 2, CH), jnp.int32),
        pltpu.VMEM((CH, DEGW), _f32),
        pltpu.VMEM_SHARED((NPAD, DEGW), _f32),
    ])
def _sc_degree(dsti_h, ones_h, z_h, out0, out1, dstv, onesv, table):
    c = lax.axis_index("c")
    s = lax.axis_index("s")
    half = CHUNKS // 2
    pltpu.sync_copy(ones_h, onesv)
    pltpu.sync_copy(z_h, table.at[pl.ds(s * RPT, RPT)])
    plsc.subcore_barrier()

    for bi in range(2):
        pltpu.sync_copy(dsti_h.at[s].at[pl.ds(bi * half, half)], dstv)

        @pl.when(c == bi)
        def _():
            @pl.loop(0, half)
            def _(j):
                pltpu.sync_copy(onesv, table.at[dstv.at[j]], add=True)

    plsc.subcore_barrier()

    @pl.when(c == 0)
    def _():
        pltpu.sync_copy(table.at[pl.ds(s * RPT, RPT)], out0.at[pl.ds(s * RPT, RPT)])

    @pl.when(c == 1)
    def _():
        pltpu.sync_copy(table.at[pl.ds(s * RPT, RPT)], out1.at[pl.ds(s * RPT, RPT)])


IDX_BLK = CHUNKS // 2


def _make_sc_agg(D):
    @functools.partial(
        pl.kernel, mesh=_mesh,
        out_type=(_sds((NPAD, D)), _sds((NPAD, D))),
        scratch_types=[
            pltpu.VMEM((IDX_BLK, CH), jnp.int32),
            pltpu.VMEM((IDX_BLK, CH), jnp.int32),
            pltpu.VMEM((CH, D), _f32),
            pltpu.VMEM_SHARED((NPAD, D), _f32),
        ])
    def kagg(h0, h1, srci, dsti, z_h, out0, out1, srcv, dstv, gbuf, table):
        c = lax.axis_index("c")
        s = lax.axis_index("s")
        pltpu.sync_copy(z_h, table.at[pl.ds(s * RPT, RPT)])
        plsc.subcore_barrier()

        def run(h):
            for bi in range(CHUNKS // IDX_BLK):
                pltpu.sync_copy(srci.at[s].at[pl.ds(bi * IDX_BLK, IDX_BLK)], srcv)
                pltpu.sync_copy(dsti.at[s].at[pl.ds(bi * IDX_BLK, IDX_BLK)], dstv)

                @pl.loop(0, IDX_BLK)
                def _(j):
                    pltpu.sync_copy(h.at[srcv.at[j]], gbuf)
                    pltpu.sync_copy(gbuf, table.at[dstv.at[j]], add=True)

        @pl.when(c == 0)
        def _():
            run(h0)

        @pl.when(c == 1)
        def _():
            run(h1)

        plsc.subcore_barrier()

        @pl.when(c == 0)
        def _():
            pltpu.sync_copy(table.at[pl.ds(s * RPT, RPT)], out0.at[pl.ds(s * RPT, RPT)])

        @pl.when(c == 1)
        def _():
            pltpu.sync_copy(table.at[pl.ds(s * RPT, RPT)], out1.at[pl.ds(s * RPT, RPT)])

    return kagg


_sc_agg128 = _make_sc_agg(HIDDEN // 2)


_BLK = 1000
_GRID = N_NODES // _BLK


def _tc_mm1(x, W1):
    H2 = HIDDEN // 2

    def body(x_ref, w_ref, oa, ob):
        h = jnp.dot(x_ref[...], w_ref[...], preferred_element_type=_f32)
        oa[...] = h[:, :H2]
        ob[...] = h[:, H2:]

    return pl.pallas_call(
        body, grid=(_GRID,),
        in_specs=[pl.BlockSpec((_BLK, D_FEAT), lambda i: (i, 0)),
                  pl.BlockSpec((D_FEAT, HIDDEN), lambda i: (0, 0))],
        out_specs=[pl.BlockSpec((_BLK, H2), lambda i: (i, 0))] * 2,
        out_shape=(_sds((N_NODES, H2)),) * 2,
    )(x, W1)


def _tc_scale(h1a, h1b, dega, degb):
    H2 = HIDDEN // 2

    def body(ha, hb, da, db, dinv_ref, oa, ob):
        d = da[...][:, 0:1] + db[...][:, 0:1] + 1.0
        dv = lax.rsqrt(d)
        dinv_ref[...] = dv
        oa[...] = ha[...] * dv
        ob[...] = hb[...] * dv

    return pl.pallas_call(
        body, grid=(_GRID,),
        in_specs=[pl.BlockSpec((_BLK, H2), lambda i: (i, 0)),
                  pl.BlockSpec((_BLK, H2), lambda i: (i, 0)),
                  pl.BlockSpec((_BLK, DEGW), lambda i: (i, 0)),
                  pl.BlockSpec((_BLK, DEGW), lambda i: (i, 0))],
        out_specs=[pl.BlockSpec((_BLK, 1), lambda i: (i, 0)),
                   pl.BlockSpec((_BLK, H2), lambda i: (i, 0)),
                   pl.BlockSpec((_BLK, H2), lambda i: (i, 0))],
        out_shape=(_sds((N_NODES, 1)), _sds((N_NODES, H2)), _sds((N_NODES, H2))),
    )(h1a, h1b, dega, degb)


def _tc_fuse(aa, ab, hsa, hsb, dinv, b, W, split):
    H2 = HIDDEN // 2
    wN = W.shape[1]
    out_half = wN // 2

    def body(aa_r, ab_r, hsa_r, hsb_r, dv_r, b_r, w_r, *outs):
        dv = dv_r[...]
        z = jnp.concatenate([aa_r[...] + hsa_r[...], ab_r[...] + hsb_r[...]], axis=1)
        z = z * dv + b_r[...]
        z = jnp.maximum(z, 0.0)
        h = jnp.dot(z, w_r[...], preferred_element_type=_f32) * dv
        if split:
            outs[0][...] = h[:, :out_half]
            outs[1][...] = h[:, out_half:]
        else:
            outs[0][...] = h

    if split:
        out_specs = [pl.BlockSpec((_BLK, out_half), lambda i: (i, 0))] * 2
        out_shape = (_sds((N_NODES, out_half)),) * 2
    else:
        out_specs = [pl.BlockSpec((_BLK, wN), lambda i: (i, 0))]
        out_shape = (_sds((N_NODES, wN)),)

    return pl.pallas_call(
        body, grid=(_GRID,),
        in_specs=[pl.BlockSpec((_BLK, H2), lambda i: (i, 0)),
                  pl.BlockSpec((_BLK, H2), lambda i: (i, 0)),
                  pl.BlockSpec((_BLK, H2), lambda i: (i, 0)),
                  pl.BlockSpec((_BLK, H2), lambda i: (i, 0)),
                  pl.BlockSpec((_BLK, 1), lambda i: (i, 0)),
                  pl.BlockSpec((1, HIDDEN), lambda i: (0, 0)),
                  pl.BlockSpec((HIDDEN, wN), lambda i: (0, 0))],
        out_specs=out_specs,
        out_shape=out_shape,
    )(aa, ab, hsa, hsb, dinv, b.reshape(1, HIDDEN), W)


def _tc_final(a3, h3s, dinv, b_out):
    def body(a_r, hs_r, dv_r, b_r, o):
        o[...] = (a_r[...] + hs_r[...]) * dv_r[...] + b_r[...]

    return pl.pallas_call(
        body, grid=(_GRID,),
        in_specs=[pl.BlockSpec((_BLK, D_OUT), lambda i: (i, 0)),
                  pl.BlockSpec((_BLK, D_OUT), lambda i: (i, 0)),
                  pl.BlockSpec((_BLK, 1), lambda i: (i, 0)),
                  pl.BlockSpec((1, D_OUT), lambda i: (0, 0))],
        out_specs=pl.BlockSpec((_BLK, D_OUT), lambda i: (i, 0)),
        out_shape=_sds((N_NODES, D_OUT)),
    )(a3, h3s, dinv, b_out.reshape(1, D_OUT))


def kernel(x, edge_index, batch, W1, b1, W2, b2, W_out, b_out):
    src = edge_index[0]
    dst = edge_index[1]
    pad = EDGES_PAD - N_EDGES
    srcp = jnp.concatenate([src, jnp.zeros((pad,), jnp.int32)]).reshape(NT, CHUNKS, CH)
    dstp = jnp.concatenate([dst, jnp.full((pad,), DUMMY, jnp.int32)]).reshape(NT, CHUNKS, CH)
    ones16 = jnp.ones((CH, DEGW), _f32)
    z16 = jnp.zeros((RPT, DEGW), _f32)
    z128 = jnp.zeros((RPT, HIDDEN // 2), _f32)

    dega, degb = _sc_degree(dstp, ones16, z16)
    h1a, h1b = _tc_mm1(x, W1)
    dinv, h1sa, h1sb = _tc_scale(h1a, h1b, dega, degb)
    a1a, a1b = _sc_agg128(h1sa, h1sb, srcp, dstp, z128)
    h2sa, h2sb = _tc_fuse(a1a, a1b, h1sa, h1sb, dinv, b1, W2, split=True)
    a2a, a2b = _sc_agg128(h2sa, h2sb, srcp, dstp, z128)
    (h3s,) = _tc_fuse(a2a, a2b, h2sa, h2sb, dinv, b2, W_out, split=False)
    a3, _ = _sc_agg128(h3s, h3s, srcp, dstp, z128)
    return _tc_final(a3, h3s, dinv, b_out)


# speedup vs baseline: 4.7904x; 1.0981x over previous
import functools

import jax
import jax.numpy as jnp
from jax import lax
from jax.experimental import pallas as pl
from jax.experimental.pallas import tpu as pltpu
from jax.experimental.pallas import tpu_sc as plsc

N_NODES = 10000
N_EDGES = 320000
D_FEAT = 128
HIDDEN = 256
D_OUT = 128

NT = 16
CH = 128
CHUNKS = 160
EDGES_PAD = NT * CHUNKS * CH
NPAD = 10240
RPT = NPAD // NT
DUMMY = N_NODES
DEGW = 128

_f32 = jnp.float32
_mesh = plsc.VectorSubcoreMesh(core_axis_name="c", subcore_axis_name="s")


def _sds(shape):
    return jax.ShapeDtypeStruct(shape, _f32)


@functools.partial(
    pl.kernel, mesh=_mesh,
    out_type=(_sds((NPAD, DEGW)), _sds((NPAD, DEGW))),
    scratch_types=[
        pltpu.VMEM((CHUNKS // 2, CH), jnp.int32),
        pltpu.VMEM((CH, DEGW), _f32),
        pltpu.VMEM_SHARED((NPAD, DEGW), _f32),
    ])
def _sc_degree(dsti_h, ones_h, z_h, out0, out1, dstv, onesv, table):
    c = lax.axis_index("c")
    s = lax.axis_index("s")
    half = CHUNKS // 2
    pltpu.sync_copy(ones_h, onesv)
    pltpu.sync_copy(z_h, table.at[pl.ds(s * RPT, RPT)])
    plsc.subcore_barrier()

    for bi in range(2):
        pltpu.sync_copy(dsti_h.at[s].at[pl.ds(bi * half, half)], dstv)

        @pl.when(c == bi)
        def _():
            @pl.loop(0, half)
            def _(j):
                pltpu.sync_copy(onesv, table.at[dstv.at[j]], add=True)

    plsc.subcore_barrier()

    @pl.when(c == 0)
    def _():
        pltpu.sync_copy(table.at[pl.ds(s * RPT, RPT)], out0.at[pl.ds(s * RPT, RPT)])

    @pl.when(c == 1)
    def _():
        pltpu.sync_copy(table.at[pl.ds(s * RPT, RPT)], out1.at[pl.ds(s * RPT, RPT)])


IDX_BLK = CHUNKS // 4


def _make_sc_agg(D):
    @functools.partial(
        pl.kernel, mesh=_mesh,
        out_type=(_sds((NPAD, D)), _sds((NPAD, D))),
        scratch_types=[
            pltpu.VMEM((IDX_BLK, CH), jnp.int32),
            pltpu.VMEM((IDX_BLK, CH), jnp.int32),
            pltpu.VMEM((CH, D), _f32),
            pltpu.VMEM((CH, D), _f32),
            pltpu.VMEM_SHARED((NPAD, D), _f32),
            pltpu.SemaphoreType.DMA,
            pltpu.SemaphoreType.DMA,
        ])
    def kagg(h0, h1, srci, dsti, z_h, out0, out1,
             srcv, dstv, gbuf0, gbuf1, table, gsem0, gsem1):
        c = lax.axis_index("c")
        s = lax.axis_index("s")
        pltpu.sync_copy(z_h, table.at[pl.ds(s * RPT, RPT)])
        plsc.subcore_barrier()

        def run(h):
            for bi in range(CHUNKS // IDX_BLK):
                pltpu.sync_copy(srci.at[s].at[pl.ds(bi * IDX_BLK, IDX_BLK)], srcv)
                pltpu.sync_copy(dsti.at[s].at[pl.ds(bi * IDX_BLK, IDX_BLK)], dstv)
                pltpu.async_copy(h.at[srcv.at[0]], gbuf0, gsem0)

                @pl.loop(0, IDX_BLK // 2)
                def _(p):
                    j0 = 2 * p
                    pltpu.make_async_copy(h.at[srcv.at[j0]], gbuf0, gsem0).wait()
                    pltpu.async_copy(h.at[srcv.at[j0 + 1]], gbuf1, gsem1)
                    pltpu.sync_copy(gbuf0, table.at[dstv.at[j0]], add=True)
                    pltpu.make_async_copy(h.at[srcv.at[j0 + 1]], gbuf1, gsem1).wait()

                    @pl.when(p < IDX_BLK // 2 - 1)
                    def _():
                        pltpu.async_copy(h.at[srcv.at[j0 + 2]], gbuf0, gsem0)

                    pltpu.sync_copy(gbuf1, table.at[dstv.at[j0 + 1]], add=True)

        @pl.when(c == 0)
        def _():
            run(h0)

        @pl.when(c == 1)
        def _():
            run(h1)

        plsc.subcore_barrier()

        @pl.when(c == 0)
        def _():
            pltpu.sync_copy(table.at[pl.ds(s * RPT, RPT)], out0.at[pl.ds(s * RPT, RPT)])

        @pl.when(c == 1)
        def _():
            pltpu.sync_copy(table.at[pl.ds(s * RPT, RPT)], out1.at[pl.ds(s * RPT, RPT)])

    return kagg


_sc_agg128 = _make_sc_agg(HIDDEN // 2)


_BLK = 1000
_GRID = N_NODES // _BLK


def _tc_mm1(x, W1):
    H2 = HIDDEN // 2

    def body(x_ref, w_ref, oa, ob):
        h = jnp.dot(x_ref[...], w_ref[...], preferred_element_type=_f32)
        oa[...] = h[:, :H2]
        ob[...] = h[:, H2:]

    return pl.pallas_call(
        body, grid=(_GRID,),
        in_specs=[pl.BlockSpec((_BLK, D_FEAT), lambda i: (i, 0)),
                  pl.BlockSpec((D_FEAT, HIDDEN), lambda i: (0, 0))],
        out_specs=[pl.BlockSpec((_BLK, H2), lambda i: (i, 0))] * 2,
        out_shape=(_sds((N_NODES, H2)),) * 2,
    )(x, W1)


def _tc_scale(h1a, h1b, dega, degb):
    H2 = HIDDEN // 2

    def body(ha, hb, da, db, dinv_ref, oa, ob):
        d = da[...][:, 0:1] + db[...][:, 0:1] + 1.0
        dv = lax.rsqrt(d)
        dinv_ref[...] = dv
        oa[...] = ha[...] * dv
        ob[...] = hb[...] * dv

    return pl.pallas_call(
        body, grid=(_GRID,),
        in_specs=[pl.BlockSpec((_BLK, H2), lambda i: (i, 0)),
                  pl.BlockSpec((_BLK, H2), lambda i: (i, 0)),
                  pl.BlockSpec((_BLK, DEGW), lambda i: (i, 0)),
                  pl.BlockSpec((_BLK, DEGW), lambda i: (i, 0))],
        out_specs=[pl.BlockSpec((_BLK, 1), lambda i: (i, 0)),
                   pl.BlockSpec((_BLK, H2), lambda i: (i, 0)),
                   pl.BlockSpec((_BLK, H2), lambda i: (i, 0))],
        out_shape=(_sds((N_NODES, 1)), _sds((N_NODES, H2)), _sds((N_NODES, H2))),
    )(h1a, h1b, dega, degb)


def _tc_fuse(aa, ab, hsa, hsb, dinv, b, W, split):
    H2 = HIDDEN // 2
    wN = W.shape[1]
    out_half = wN // 2

    def body(aa_r, ab_r, hsa_r, hsb_r, dv_r, b_r, w_r, *outs):
        dv = dv_r[...]
        z = jnp.concatenate([aa_r[...] + hsa_r[...], ab_r[...] + hsb_r[...]], axis=1)
        z = z * dv + b_r[...]
        z = jnp.maximum(z, 0.0)
        h = jnp.dot(z, w_r[...], preferred_element_type=_f32) * dv
        if split:
            outs[0][...] = h[:, :out_half]
            outs[1][...] = h[:, out_half:]
        else:
            outs[0][...] = h

    if split:
        out_specs = [pl.BlockSpec((_BLK, out_half), lambda i: (i, 0))] * 2
        out_shape = (_sds((N_NODES, out_half)),) * 2
    else:
        out_specs = [pl.BlockSpec((_BLK, wN), lambda i: (i, 0))]
        out_shape = (_sds((N_NODES, wN)),)

    return pl.pallas_call(
        body, grid=(_GRID,),
        in_specs=[pl.BlockSpec((_BLK, H2), lambda i: (i, 0)),
                  pl.BlockSpec((_BLK, H2), lambda i: (i, 0)),
                  pl.BlockSpec((_BLK, H2), lambda i: (i, 0)),
                  pl.BlockSpec((_BLK, H2), lambda i: (i, 0)),
                  pl.BlockSpec((_BLK, 1), lambda i: (i, 0)),
                  pl.BlockSpec((1, HIDDEN), lambda i: (0, 0)),
                  pl.BlockSpec((HIDDEN, wN), lambda i: (0, 0))],
        out_specs=out_specs,
        out_shape=out_shape,
    )(aa, ab, hsa, hsb, dinv, b.reshape(1, HIDDEN), W)


def _tc_final(a3, h3s, dinv, b_out):
    def body(a_r, hs_r, dv_r, b_r, o):
        o[...] = (a_r[...] + hs_r[...]) * dv_r[...] + b_r[...]

    return pl.pallas_call(
        body, grid=(_GRID,),
        in_specs=[pl.BlockSpec((_BLK, D_OUT), lambda i: (i, 0)),
                  pl.BlockSpec((_BLK, D_OUT), lambda i: (i, 0)),
                  pl.BlockSpec((_BLK, 1), lambda i: (i, 0)),
                  pl.BlockSpec((1, D_OUT), lambda i: (0, 0))],
        out_specs=pl.BlockSpec((_BLK, D_OUT), lambda i: (i, 0)),
        out_shape=_sds((N_NODES, D_OUT)),
    )(a3, h3s, dinv, b_out.reshape(1, D_OUT))


def kernel(x, edge_index, batch, W1, b1, W2, b2, W_out, b_out):
    src = edge_index[0]
    dst = edge_index[1]
    pad = EDGES_PAD - N_EDGES
    srcp = jnp.concatenate([src, jnp.zeros((pad,), jnp.int32)]).reshape(NT, CHUNKS, CH)
    dstp = jnp.concatenate([dst, jnp.full((pad,), DUMMY, jnp.int32)]).reshape(NT, CHUNKS, CH)
    ones16 = jnp.ones((CH, DEGW), _f32)
    z16 = jnp.zeros((RPT, DEGW), _f32)
    z128 = jnp.zeros((RPT, HIDDEN // 2), _f32)

    dega, degb = _sc_degree(dstp, ones16, z16)
    h1a, h1b = _tc_mm1(x, W1)
    dinv, h1sa, h1sb = _tc_scale(h1a, h1b, dega, degb)
    a1a, a1b = _sc_agg128(h1sa, h1sb, srcp, dstp, z128)
    h2sa, h2sb = _tc_fuse(a1a, a1b, h1sa, h1sb, dinv, b1, W2, split=True)
    a2a, a2b = _sc_agg128(h2sa, h2sb, srcp, dstp, z128)
    (h3s,) = _tc_fuse(a2a, a2b, h2sa, h2sb, dinv, b2, W_out, split=False)
    a3, _ = _sc_agg128(h3s, h3s, srcp, dstp, z128)
    return _tc_final(a3, h3s, dinv, b_out)


# speedup vs baseline: 4.7906x; 1.0000x over previous
import functools

import jax
import jax.numpy as jnp
from jax import lax
from jax.experimental import pallas as pl
from jax.experimental.pallas import tpu as pltpu
from jax.experimental.pallas import tpu_sc as plsc

N_NODES = 10000
N_EDGES = 320000
D_FEAT = 128
HIDDEN = 256
D_OUT = 128

NT = 16
CH = 128
CHUNKS = 160
EDGES_PAD = NT * CHUNKS * CH
NPAD = 10240
RPT = NPAD // NT
DUMMY = N_NODES
DEGW = 128

_f32 = jnp.float32
_mesh = plsc.VectorSubcoreMesh(core_axis_name="c", subcore_axis_name="s")


def _sds(shape):
    return jax.ShapeDtypeStruct(shape, _f32)


@functools.partial(
    pl.kernel, mesh=_mesh,
    out_type=(_sds((NPAD, DEGW)), _sds((NPAD, DEGW))),
    scratch_types=[
        pltpu.VMEM((CHUNKS // 2, CH), jnp.int32),
        pltpu.VMEM((CH, DEGW), _f32),
        pltpu.VMEM_SHARED((NPAD, DEGW), _f32),
    ])
def _sc_degree(dsti_h, ones_h, z_h, out0, out1, dstv, onesv, table):
    c = lax.axis_index("c")
    s = lax.axis_index("s")
    half = CHUNKS // 2
    pltpu.sync_copy(ones_h, onesv)
    pltpu.sync_copy(z_h, table.at[pl.ds(s * RPT, RPT)])
    plsc.subcore_barrier()

    for bi in range(2):
        pltpu.sync_copy(dsti_h.at[s].at[pl.ds(bi * half, half)], dstv)

        @pl.when(c == bi)
        def _():
            @pl.loop(0, half)
            def _(j):
                pltpu.sync_copy(onesv, table.at[dstv.at[j]], add=True)

    plsc.subcore_barrier()

    @pl.when(c == 0)
    def _():
        pltpu.sync_copy(table.at[pl.ds(s * RPT, RPT)], out0.at[pl.ds(s * RPT, RPT)])

    @pl.when(c == 1)
    def _():
        pltpu.sync_copy(table.at[pl.ds(s * RPT, RPT)], out1.at[pl.ds(s * RPT, RPT)])


IDX_BLK = CHUNKS // 4


def _make_sc_agg(D, chunks, idx_blk):
    scratch = [
        pltpu.VMEM((idx_blk, CH), jnp.int32),
        pltpu.VMEM((idx_blk, CH), jnp.int32),
        pltpu.VMEM((CH, D), _f32),
        pltpu.VMEM((CH, D), _f32),
        pltpu.VMEM_SHARED((NPAD, D), _f32),
        pltpu.SemaphoreType.DMA,
        pltpu.SemaphoreType.DMA,
    ]

    def kagg_impl(h0, h1, srci, dsti, z_h, out0, out1,
                  srcv, dstv, gbuf0, gbuf1, table, gsem0, gsem1):
        c = lax.axis_index("c")
        s = lax.axis_index("s")
        pltpu.sync_copy(z_h, table.at[pl.ds(s * RPT, RPT)])
        plsc.subcore_barrier()

        def run(h, src_block, dst_block):
            for bi in range(chunks // idx_blk):
                pltpu.sync_copy(src_block(bi), srcv)
                pltpu.sync_copy(dst_block(bi), dstv)
                pltpu.async_copy(h.at[srcv.at[0]], gbuf0, gsem0)

                @pl.loop(0, idx_blk // 2)
                def _(p):
                    j0 = 2 * p
                    pltpu.make_async_copy(h.at[srcv.at[j0]], gbuf0, gsem0).wait()
                    pltpu.async_copy(h.at[srcv.at[j0 + 1]], gbuf1, gsem1)
                    pltpu.sync_copy(gbuf0, table.at[dstv.at[j0]], add=True)
                    pltpu.make_async_copy(h.at[srcv.at[j0 + 1]], gbuf1, gsem1).wait()

                    @pl.when(p < idx_blk // 2 - 1)
                    def _():
                        pltpu.async_copy(h.at[srcv.at[j0 + 2]], gbuf0, gsem0)

                    pltpu.sync_copy(gbuf1, table.at[dstv.at[j0 + 1]], add=True)

        @pl.when(c == 0)
        def _():
            run(h0,
                lambda bi: srci.at[s].at[pl.ds(bi * idx_blk, idx_blk)],
                lambda bi: dsti.at[s].at[pl.ds(bi * idx_blk, idx_blk)])

        @pl.when(c == 1)
        def _():
            run(h1,
                lambda bi: srci.at[s].at[pl.ds(bi * idx_blk, idx_blk)],
                lambda bi: dsti.at[s].at[pl.ds(bi * idx_blk, idx_blk)])

        plsc.subcore_barrier()

        @pl.when(c == 0)
        def _():
            pltpu.sync_copy(table.at[pl.ds(s * RPT, RPT)], out0.at[pl.ds(s * RPT, RPT)])

        @pl.when(c == 1)
        def _():
            pltpu.sync_copy(table.at[pl.ds(s * RPT, RPT)], out1.at[pl.ds(s * RPT, RPT)])

    @functools.partial(
        pl.kernel, mesh=_mesh,
        out_type=(_sds((NPAD, D)), _sds((NPAD, D))),
        scratch_types=scratch)
    def kagg(h0, h1, srci, dsti, z_h, out0, out1, *rest):
        kagg_impl(h0, h1, srci, dsti, z_h, out0, out1, *rest)

    return kagg


_sc_agg128 = _make_sc_agg(HIDDEN // 2, CHUNKS, IDX_BLK)


_BLK = 1000
_GRID = N_NODES // _BLK


def _tc_mm1(x, W1):
    H2 = HIDDEN // 2

    def body(x_ref, w_ref, oa, ob):
        h = jnp.dot(x_ref[...], w_ref[...], preferred_element_type=_f32)
        oa[...] = h[:, :H2]
        ob[...] = h[:, H2:]

    return pl.pallas_call(
        body, grid=(_GRID,),
        in_specs=[pl.BlockSpec((_BLK, D_FEAT), lambda i: (i, 0)),
                  pl.BlockSpec((D_FEAT, HIDDEN), lambda i: (0, 0))],
        out_specs=[pl.BlockSpec((_BLK, H2), lambda i: (i, 0))] * 2,
        out_shape=(_sds((N_NODES, H2)),) * 2,
    )(x, W1)


def _tc_scale(h1a, h1b, dega, degb):
    H2 = HIDDEN // 2

    def body(ha, hb, da, db, dinv_ref, oa, ob):
        d = da[...][:, 0:1] + db[...][:, 0:1] + 1.0
        dv = lax.rsqrt(d)
        dinv_ref[...] = dv
        oa[...] = ha[...] * dv
        ob[...] = hb[...] * dv

    return pl.pallas_call(
        body, grid=(_GRID,),
        in_specs=[pl.BlockSpec((_BLK, H2), lambda i: (i, 0)),
                  pl.BlockSpec((_BLK, H2), lambda i: (i, 0)),
                  pl.BlockSpec((_BLK, DEGW), lambda i: (i, 0)),
                  pl.BlockSpec((_BLK, DEGW), lambda i: (i, 0))],
        out_specs=[pl.BlockSpec((_BLK, 1), lambda i: (i, 0)),
                   pl.BlockSpec((_BLK, H2), lambda i: (i, 0)),
                   pl.BlockSpec((_BLK, H2), lambda i: (i, 0))],
        out_shape=(_sds((N_NODES, 1)), _sds((N_NODES, H2)), _sds((N_NODES, H2))),
    )(h1a, h1b, dega, degb)


def _tc_fuse(aa, ab, hsa, hsb, dinv, b, W, split):
    H2 = HIDDEN // 2
    wN = W.shape[1]
    out_half = wN // 2

    def body(aa_r, ab_r, hsa_r, hsb_r, dv_r, b_r, w_r, *outs):
        dv = dv_r[...]
        z = jnp.concatenate([aa_r[...] + hsa_r[...], ab_r[...] + hsb_r[...]], axis=1)
        z = z * dv + b_r[...]
        z = jnp.maximum(z, 0.0)
        h = jnp.dot(z, w_r[...], preferred_element_type=_f32) * dv
        if split:
            outs[0][...] = h[:, :out_half]
            outs[1][...] = h[:, out_half:]
        else:
            outs[0][...] = h

    if split:
        out_specs = [pl.BlockSpec((_BLK, out_half), lambda i: (i, 0))] * 2
        out_shape = (_sds((N_NODES, out_half)),) * 2
    else:
        out_specs = [pl.BlockSpec((_BLK, wN), lambda i: (i, 0))]
        out_shape = (_sds((N_NODES, wN)),)

    return pl.pallas_call(
        body, grid=(_GRID,),
        in_specs=[pl.BlockSpec((_BLK, H2), lambda i: (i, 0)),
                  pl.BlockSpec((_BLK, H2), lambda i: (i, 0)),
                  pl.BlockSpec((_BLK, H2), lambda i: (i, 0)),
                  pl.BlockSpec((_BLK, H2), lambda i: (i, 0)),
                  pl.BlockSpec((_BLK, 1), lambda i: (i, 0)),
                  pl.BlockSpec((1, HIDDEN), lambda i: (0, 0)),
                  pl.BlockSpec((HIDDEN, wN), lambda i: (0, 0))],
        out_specs=out_specs,
        out_shape=out_shape,
    )(aa, ab, hsa, hsb, dinv, b.reshape(1, HIDDEN), W)


def _tc_final(a3, h3s, dinv, b_out):
    def body(a_r, hs_r, dv_r, b_r, o):
        o[...] = (a_r[...] + hs_r[...]) * dv_r[...] + b_r[...]

    return pl.pallas_call(
        body, grid=(_GRID,),
        in_specs=[pl.BlockSpec((_BLK, D_OUT), lambda i: (i, 0)),
                  pl.BlockSpec((_BLK, D_OUT), lambda i: (i, 0)),
                  pl.BlockSpec((_BLK, 1), lambda i: (i, 0)),
                  pl.BlockSpec((1, D_OUT), lambda i: (0, 0))],
        out_specs=pl.BlockSpec((_BLK, D_OUT), lambda i: (i, 0)),
        out_shape=_sds((N_NODES, D_OUT)),
    )(a3, h3s, dinv, b_out.reshape(1, D_OUT))


def kernel(x, edge_index, batch, W1, b1, W2, b2, W_out, b_out):
    src = edge_index[0]
    dst = edge_index[1]
    pad = EDGES_PAD - N_EDGES
    srcp = jnp.concatenate([src, jnp.zeros((pad,), jnp.int32)]).reshape(NT, CHUNKS, CH)
    dstp = jnp.concatenate([dst, jnp.full((pad,), DUMMY, jnp.int32)]).reshape(NT, CHUNKS, CH)
    ones16 = jnp.ones((CH, DEGW), _f32)
    z16 = jnp.zeros((RPT, DEGW), _f32)
    z128 = jnp.zeros((RPT, HIDDEN // 2), _f32)

    dega, degb = _sc_degree(dstp, ones16, z16)
    h1a, h1b = _tc_mm1(x, W1)
    dinv, h1sa, h1sb = _tc_scale(h1a, h1b, dega, degb)
    a1a, a1b = _sc_agg128(h1sa, h1sb, srcp, dstp, z128)
    h2sa, h2sb = _tc_fuse(a1a, a1b, h1sa, h1sb, dinv, b1, W2, split=True)
    a2a, a2b = _sc_agg128(h2sa, h2sb, srcp, dstp, z128)
    (h3s,) = _tc_fuse(a2a, a2b, h2sa, h2sb, dinv, b2, W_out, split=False)
    a3, _ = _sc_agg128(h3s, h3s, srcp, dstp, z128)
    return _tc_final(a3, h3s, dinv, b_out)
